# baseline (speedup 1.0000x reference)
import functools

import jax
import jax.numpy as jnp
from jax import lax
from jax.experimental import pallas as pl
from jax.experimental.pallas import tpu as pltpu
from jax.experimental.pallas import tpu_sc as plsc

_D = 128
_B = 16384
_NC = 2
_NS = 16
_NW = _NC * _NS
_BPW = _B // _NW
_HV = 128
_CH = 128
_NG = (_BPW - _HV) // _CH
_NJ = _D // 16

_mesh = plsc.VectorSubcoreMesh(core_axis_name="c", subcore_axis_name="s")


@functools.partial(
    pl.kernel,
    mesh=_mesh,
    out_type=jax.ShapeDtypeStruct((1, _B, _D), jnp.float32),
    scratch_types=[
        pltpu.VMEM_SHARED((_NS, 2, _D), jnp.float32),
        pltpu.VMEM((2 * _D,), jnp.float32),
        pltpu.VMEM((_BPW,), jnp.int32),
        pltpu.VMEM((_BPW, _D), jnp.float32),
    ] + [pltpu.SemaphoreType.DMA] * (5 + 2 * _NG + 1),
)
def _select_kernel(pres_hbm, abs_hbm, idx_hbm, out_hbm,
                   table_s, table_v, flags_v, rows_v,
                   sem_p, sem_a, sem_lp, sem_la, sem_f, *osem):
    cid = lax.axis_index("c")
    sid = lax.axis_index("s")
    wid = sid * _NC + cid
    base = wid * _BPW
    out2d = out_hbm.at[0]
    gsem = list(osem[:_NG])
    sem_o = osem[_NG]
    csem = list(osem[_NG + 1:])
    cp_p = pltpu.async_copy(pres_hbm, table_s.at[sid].at[0], sem_p)
    cp_a = pltpu.async_copy(abs_hbm, table_s.at[sid].at[1], sem_a)
    cp_lp = pltpu.async_copy(pres_hbm, table_v.at[pl.ds(0, _D)], sem_lp)
    cp_la = pltpu.async_copy(abs_hbm, table_v.at[pl.ds(_D, _D)], sem_la)
    cp_f = pltpu.async_copy(idx_hbm.at[pl.ds(base, _BPW)], flags_v, sem_f)
    cp_p.wait()
    cp_a.wait()
    cp_f.wait()
    gaths = []
    for k in range(_NG):
        off = _HV + k * _CH
        gaths.append(pltpu.async_copy(
            table_s.at[sid].at[flags_v.at[pl.ds(off, _CH)]],
            rows_v.at[pl.ds(off, _CH)], gsem[k]))
    cp_lp.wait()
    cp_la.wait()
    pres = [table_v[pl.ds(16 * j, 16)] for j in range(_NJ)]
    diff = [table_v[pl.ds(_D + 16 * j, 16)] - pres[j] for j in range(_NJ)]
    lane = [jnp.full((16, 1), l, jnp.int32) for l in range(16)]
    dnums = lax.GatherDimensionNumbers(
        offset_dims=(), collapsed_slice_dims=(0,), start_index_map=(0,))

    def body(g, carry):
        fv = flags_v[pl.ds(g * 16, 16)]
        rbase = g * 16
        for l in range(16):
            bl = lax.gather(fv, lane[l], dnums, (1,),
                            mode=lax.GatherScatterMode.PROMISE_IN_BOUNDS)
            f = bl.astype(jnp.float32)
            for j in range(_NJ):
                rows_v[rbase + l, pl.ds(16 * j, 16)] = pres[j] + f * diff[j]
        return carry

    lax.fori_loop(0, _HV // 16, body, 0)
    cp_o = pltpu.async_copy(rows_v.at[pl.ds(0, _HV)],
                            out2d.at[pl.ds(base, _HV)], sem_o)
    outs = []
    for k in range(_NG):
        off = _HV + k * _CH
        gaths[k].wait()
        outs.append(pltpu.async_copy(
            rows_v.at[pl.ds(off, _CH)],
            out2d.at[pl.ds(base + off, _CH)], csem[k % len(csem)]))
    cp_o.wait()
    for o in outs:
        o.wait()


def kernel(is_absent, present_goal_vector, absent_goal_vector):
    idx = is_absent.astype(jnp.int32)
    return _select_kernel(present_goal_vector, absent_goal_vector, idx)


# speedup vs baseline: 1.1135x; 1.1135x over previous
import functools

import jax
import jax.numpy as jnp
from jax import lax
from jax.experimental import pallas as pl
from jax.experimental.pallas import tpu as pltpu
from jax.experimental.pallas import tpu_sc as plsc

_D = 128
_B = 16384
_NC = 2
_NS = 16
_NW = _NC * _NS
_BPW = _B // _NW
_NCH = 4
_CH = _BPW // _NCH

_mesh = plsc.VectorSubcoreMesh(core_axis_name="c", subcore_axis_name="s")


@functools.partial(
    pl.kernel,
    mesh=_mesh,
    out_type=jax.ShapeDtypeStruct((1, _B, _D), jnp.float32),
    scratch_types=[
        pltpu.VMEM_SHARED((_NS, 2, _D), jnp.float32),
        pltpu.VMEM((_BPW,), jnp.int32),
        pltpu.VMEM((_BPW, _D), jnp.float32),
    ] + [pltpu.SemaphoreType.DMA] * (2 * _NCH + 3),
)
def _gather_kernel(pres_hbm, abs_hbm, idx_hbm, out_hbm,
                   table_s, flags_v, rows_v, sem_p, sem_a, sem_o, *ksem):
    cid = lax.axis_index("c")
    sid = lax.axis_index("s")
    wid = sid * _NC + cid
    base = wid * _BPW
    out2d = out_hbm.at[0]
    sem_f = ksem[0]
    gsem = list(ksem[1:])
    cp_p = pltpu.async_copy(pres_hbm, table_s.at[sid].at[0], sem_p)
    cp_a = pltpu.async_copy(abs_hbm, table_s.at[sid].at[1], sem_a)
    cp_f = pltpu.async_copy(idx_hbm.at[pl.ds(base, _BPW)], flags_v, sem_f)
    cp_p.wait()
    cp_a.wait()
    cp_f.wait()
    gaths = []
    for k in range(_NCH):
        gaths.append(pltpu.async_copy(
            table_s.at[sid].at[flags_v.at[pl.ds(k * _CH, _CH)]],
            rows_v.at[pl.ds(k * _CH, _CH)], gsem[k]))
    outs = []
    for k in range(_NCH):
        gaths[k].wait()
        outs.append(pltpu.async_copy(
            rows_v.at[pl.ds(k * _CH, _CH)],
            out2d.at[pl.ds(base + k * _CH, _CH)], sem_o))
    for o in outs:
        o.wait()


def kernel(is_absent, present_goal_vector, absent_goal_vector):
    idx = is_absent.astype(jnp.int32)
    return _gather_kernel(present_goal_vector, absent_goal_vector, idx)
